# baseline (speedup 1.0000x reference)
import functools

import jax
import jax.numpy as jnp
from jax import lax
from jax.experimental import pallas as pl
from jax.experimental.pallas import tpu as pltpu
from jax.experimental.pallas import tpu_sc as plsc

B = 64
D = 128
N = 10000
E = 320000
NC = 2
NS = 16
NW = NC * NS

SUB = 128
NSUB = 3
CHUNK = SUB * NSUB
N_ECHUNKS = 26
E_PER_W = CHUNK * N_ECHUNKS
EREM_BASE = NW * E_PER_W
EREM_BLOCKS = (E - EREM_BASE) // SUB
N_NBLOCKS = N // SUB
NREM_BASE = N_NBLOCKS * SUB
NREM = N - NREM_BASE


def _histo_update(cnt_ref, seg_vec):
    r, m = plsc.scan_count(seg_vec)
    plsc.addupdate_scatter(cnt_ref, [seg_vec], r.astype(jnp.float32), mask=m)


def _sc_body(ei_hbm, x_hbm, batch_hbm, eattr_hbm,
             nacc_out, ncnt_out, eacc_out, ecnt_out,
             batch_v, ei_v0, ei_v1, attr_v0, attr_v1, xstage_v,
             sg00, sg01, sg02, sg10, sg11, sg12, nrem_v,
             ncnt_v, ecnt_v, nacc_s, eacc_s,
             gsem0, gsem1, ssem0, ssem1, eisem0, eisem1):
    c = lax.axis_index("c")
    s = lax.axis_index("s")
    w = s * NC + c

    ei_v = (ei_v0, ei_v1)
    attr_v = (attr_v0, attr_v1)
    seg_v = ((sg00, sg01, sg02), (sg10, sg11, sg12))
    nseg_v = (sg00, sg01, sg02)
    gsem = (gsem0, gsem1)
    ssem = (ssem0, ssem1)
    eisem = (eisem0, eisem1)
    ebase = w * E_PER_W
    RPT = B // NS

    def start_gather(k, b):
        cb = ebase + k * CHUNK
        pltpu.async_copy(ei_hbm.at[pl.ds(E + cb, CHUNK)], ei_v[b], eisem[b])
        pltpu.async_copy(eattr_hbm.at[pl.ds(cb, CHUNK)], attr_v[b], gsem[b])

    def wait_gather_ei(k, b):
        cb = ebase + k * CHUNK
        pltpu.make_async_copy(ei_hbm.at[pl.ds(E + cb, CHUNK)], ei_v[b],
                              eisem[b]).wait()

    def wait_gather_attr(k, b):
        cb = ebase + k * CHUNK
        pltpu.make_async_copy(eattr_hbm.at[pl.ds(cb, CHUNK)], attr_v[b],
                              gsem[b]).wait()

    start_gather(0, 0)

    NB_IT = (N_NBLOCKS + NW - 1) // NW

    for it in range(NB_IT):
        @pl.when(w + it * NW < N_NBLOCKS)
        def _(it=it):
            base = (w + it * NW) * SUB
            pltpu.async_copy(batch_hbm.at[pl.ds(base, SUB)], nseg_v[it],
                             gsem1)
            pltpu.async_copy(x_hbm.at[pl.ds(base, SUB)],
                             attr_v1.at[pl.ds(it * SUB, SUB)], gsem1)

    @pl.when(w == NW - 1)
    def _():
        pltpu.async_copy(batch_hbm.at[pl.ds(NREM_BASE, NREM)], nrem_v, gsem1)
        pltpu.async_copy(x_hbm.at[pl.ds(NREM_BASE, NREM)],
                         xstage_v.at[pl.ds(0, NREM)], gsem1)

    zero16 = jnp.zeros((16,), jnp.float32)
    for t in range(B // 16):
        ncnt_v[pl.ds(t * 16, 16)] = zero16
        ecnt_v[pl.ds(t * 16, 16)] = zero16
    zstage = xstage_v.at[pl.ds(NREM, RPT)]
    for i in range(RPT):
        for j in range(D // 16):
            xstage_v[NREM + i, pl.ds(j * 16, 16)] = zero16

    pltpu.sync_copy(batch_hbm, batch_v)

    pltpu.sync_copy(zstage, nacc_s.at[pl.ds(s * RPT, RPT)])
    pltpu.sync_copy(zstage, eacc_s.at[pl.ds(s * RPT, RPT)])

    plsc.subcore_barrier()

    for it in range(NB_IT):
        @pl.when(w + it * NW < N_NBLOCKS)
        def _(it=it):
            base = (w + it * NW) * SUB
            pltpu.make_async_copy(batch_hbm.at[pl.ds(base, SUB)], nseg_v[it],
                                  gsem1).wait()
            pltpu.make_async_copy(x_hbm.at[pl.ds(base, SUB)],
                                  attr_v1.at[pl.ds(it * SUB, SUB)],
                                  gsem1).wait()
            pltpu.async_copy(attr_v1.at[pl.ds(it * SUB, SUB)],
                             nacc_s.at[nseg_v[it]], ssem0, add=True)
            for t in range(SUB // 16):
                _histo_update(ncnt_v, nseg_v[it][pl.ds(t * 16, 16)])

    @pl.when(w == NW - 1)
    def _():
        pltpu.make_async_copy(batch_hbm.at[pl.ds(NREM_BASE, NREM)], nrem_v,
                              gsem1).wait()
        pltpu.make_async_copy(x_hbm.at[pl.ds(NREM_BASE, NREM)],
                              xstage_v.at[pl.ds(0, NREM)], gsem1).wait()
        pltpu.async_copy(xstage_v.at[pl.ds(0, NREM)], nacc_s.at[nrem_v],
                         ssem0, add=True)
        _histo_update(ncnt_v, nrem_v[pl.ds(0, 16)])
        pltpu.make_async_copy(xstage_v.at[pl.ds(0, NREM)], nacc_s.at[nrem_v],
                              ssem0).wait()

    for it in range(NB_IT):
        @pl.when(w + it * NW < N_NBLOCKS)
        def _(it=it):
            pltpu.make_async_copy(attr_v1.at[pl.ds(it * SUB, SUB)],
                                  nacc_s.at[nseg_v[it]], ssem0).wait()

    start_gather(1, 1)

    def compute_segs(b):
        for j in range(NSUB):
            for t in range(SUB // 16):
                idx = ei_v[b][pl.ds(j * SUB + t * 16, 16)]
                seg = plsc.load_gather(batch_v, [idx])
                seg_v[b][j][pl.ds(t * 16, 16)] = seg
                _histo_update(ecnt_v, seg)

    def issue_scatter(b):
        for j in range(NSUB):
            pltpu.async_copy(attr_v[b].at[pl.ds(j * SUB, SUB)],
                             eacc_s.at[seg_v[b][j]], ssem[b], add=True)

    def drain_scatter(b):
        for j in range(NSUB):
            pltpu.make_async_copy(attr_v[b].at[pl.ds(j * SUB, SUB)],
                                  eacc_s.at[seg_v[b][j]], ssem[b]).wait()

    wait_gather_ei(0, 0)
    compute_segs(0)
    wait_gather_attr(0, 0)
    issue_scatter(0)

    NPAIR = 12

    def epair(k2, _):
        ka = 2 * k2 + 1
        wait_gather_ei(ka, 1)
        compute_segs(1)
        wait_gather_attr(ka, 1)
        issue_scatter(1)
        drain_scatter(0)
        start_gather(ka + 1, 0)

        kb = ka + 1
        wait_gather_ei(kb, 0)
        compute_segs(0)
        wait_gather_attr(kb, 0)
        issue_scatter(0)
        drain_scatter(1)
        start_gather(kb + 1, 1)
        return 0

    lax.fori_loop(0, NPAIR, epair, 0)

    wait_gather_ei(N_ECHUNKS - 1, 1)
    compute_segs(1)
    wait_gather_attr(N_ECHUNKS - 1, 1)
    issue_scatter(1)
    drain_scatter(0)
    drain_scatter(1)

    @pl.when(w < EREM_BLOCKS)
    def _():
        base = EREM_BASE + w * SUB
        pltpu.sync_copy(ei_hbm.at[pl.ds(E + base, SUB)], ei_v0.at[pl.ds(0, SUB)])
        pltpu.sync_copy(eattr_hbm.at[pl.ds(base, SUB)],
                        attr_v0.at[pl.ds(0, SUB)])
        for t in range(SUB // 16):
            idx = ei_v0[pl.ds(t * 16, 16)]
            seg = plsc.load_gather(batch_v, [idx])
            sg00[pl.ds(t * 16, 16)] = seg
            _histo_update(ecnt_v, seg)
        pltpu.sync_copy(attr_v0.at[pl.ds(0, SUB)], eacc_s.at[sg00], add=True)

    pltpu.sync_copy(ncnt_v, ncnt_out.at[c, s])
    pltpu.sync_copy(ecnt_v, ecnt_out.at[c, s])

    plsc.subcore_barrier()

    pltpu.sync_copy(nacc_s.at[pl.ds(s * RPT, RPT)],
                    nacc_out.at[c, pl.ds(s * RPT, RPT)])
    pltpu.sync_copy(eacc_s.at[pl.ds(s * RPT, RPT)],
                    eacc_out.at[c, pl.ds(s * RPT, RPT)])


_sc_segment_sums = functools.partial(
    pl.kernel,
    out_type=(
        jax.ShapeDtypeStruct((NC, B, D), jnp.float32),
        jax.ShapeDtypeStruct((NC, NS, B), jnp.float32),
        jax.ShapeDtypeStruct((NC, B, D), jnp.float32),
        jax.ShapeDtypeStruct((NC, NS, B), jnp.float32),
    ),
    mesh=plsc.VectorSubcoreMesh(core_axis_name="c", subcore_axis_name="s"),
    compiler_params=pltpu.CompilerParams(needs_layout_passes=False),
    scratch_types=[
        pltpu.VMEM((N,), jnp.int32),
        pltpu.VMEM((CHUNK,), jnp.int32),
        pltpu.VMEM((CHUNK,), jnp.int32),
        pltpu.VMEM((CHUNK, D), jnp.float32),
        pltpu.VMEM((CHUNK, D), jnp.float32),
        pltpu.VMEM((NREM + B // NS, D), jnp.float32),
    ] + [pltpu.VMEM((SUB,), jnp.int32)] * 6
    + [
        pltpu.VMEM((NREM,), jnp.int32),
        pltpu.VMEM((B,), jnp.float32),
        pltpu.VMEM((B,), jnp.float32),
        pltpu.VMEM_SHARED((B, D), jnp.float32),
        pltpu.VMEM_SHARED((B, D), jnp.float32),
        pltpu.SemaphoreType.DMA,
        pltpu.SemaphoreType.DMA,
        pltpu.SemaphoreType.DMA,
        pltpu.SemaphoreType.DMA,
        pltpu.SemaphoreType.DMA,
        pltpu.SemaphoreType.DMA,
    ],
)(_sc_body)


def _mlp_body(nacc, ncnt, eacc, ecnt, u, w1, b1, w2, b2, out):
    nsum = nacc[0] + nacc[1]
    esum = eacc[0] + eacc[1]
    ncount = jnp.sum(ncnt[...], axis=(0, 1)).reshape(B, 1)
    ecount = jnp.sum(ecnt[...], axis=(0, 1)).reshape(B, 1)
    nmean = nsum / jnp.maximum(ncount, 1.0)
    emean = esum / jnp.maximum(ecount, 1.0)
    U = w1.shape[0] - 2 * D
    h = (jnp.dot(u[...], w1[0:U], preferred_element_type=jnp.float32)
         + jnp.dot(nmean, w1[U:U + D], preferred_element_type=jnp.float32)
         + jnp.dot(emean, w1[U + D:], preferred_element_type=jnp.float32)
         + b1[...])
    h = jnp.maximum(h, 0.0)
    out[...] = jnp.dot(h, w2[...], preferred_element_type=jnp.float32) + b2[...]


def kernel(x, edge_index, edge_attr, u, batch, W1, b1, W2, b2):
    ei = edge_index if edge_index.dtype == jnp.int32 else edge_index.astype(jnp.int32)
    ei = ei.reshape(-1)
    batch_i = batch if batch.dtype == jnp.int32 else batch.astype(jnp.int32)
    nacc, ncnt, eacc, ecnt = _sc_segment_sums(ei, x, batch_i, edge_attr)

    out = pl.pallas_call(
        _mlp_body,
        out_shape=jax.ShapeDtypeStruct((B, D), jnp.float32),
    )(nacc, ncnt, eacc, ecnt, u, W1,
      b1.reshape(1, D), W2, b2.reshape(1, D))
    return out


# speedup vs baseline: 1.0505x; 1.0505x over previous
import functools

import jax
import jax.numpy as jnp
from jax import lax
from jax.experimental import pallas as pl
from jax.experimental.pallas import tpu as pltpu
from jax.experimental.pallas import tpu_sc as plsc

B = 64
D = 128
N = 10000
E = 320000
NC = 2
NS = 16
NW = NC * NS
NBUF = 3

SUB = 128
NSUB = 2
CHUNK = SUB * NSUB
N_ECHUNKS = 39
E_PER_W = CHUNK * N_ECHUNKS
EREM_BASE = NW * E_PER_W
EREM_BLOCKS = (E - EREM_BASE) // SUB
N_NBLOCKS = N // SUB
NREM_BASE = N_NBLOCKS * SUB
NREM = N - NREM_BASE


def _histo_update(cnt_ref, seg_vec):
    r, m = plsc.scan_count(seg_vec)
    plsc.addupdate_scatter(cnt_ref, [seg_vec], r.astype(jnp.float32), mask=m)


def _sc_body(ei_hbm, x_hbm, batch_hbm, eattr_hbm,
             nacc_out, ncnt_out, eacc_out, ecnt_out,
             batch_v, ei_v0, ei_v1, ei_v2, attr_v0, attr_v1, attr_v2,
             xstage_v, sg00, sg01, sg10, sg11, sg20, sg21, nrem_v,
             ncnt_v, ecnt_v, nacc_s, eacc_s,
             gsem0, gsem1, gsem2, ssem0, ssem1, ssem2,
             eisem0, eisem1, eisem2):
    c = lax.axis_index("c")
    s = lax.axis_index("s")
    w = s * NC + c

    ei_v = (ei_v0, ei_v1, ei_v2)
    attr_v = (attr_v0, attr_v1, attr_v2)
    seg_v = ((sg00, sg01), (sg10, sg11), (sg20, sg21))
    nseg_v = (sg00, sg01, sg10)
    nstage = (attr_v1.at[pl.ds(0, SUB)], attr_v1.at[pl.ds(SUB, SUB)],
              attr_v2.at[pl.ds(0, SUB)])
    gsem = (gsem0, gsem1, gsem2)
    ssem = (ssem0, ssem1, ssem2)
    eisem = (eisem0, eisem1, eisem2)
    ebase = w * E_PER_W
    RPT = B // NS

    def start_gather(k, b):
        cb = ebase + k * CHUNK
        pltpu.async_copy(ei_hbm.at[pl.ds(E + cb, CHUNK)], ei_v[b], eisem[b])
        pltpu.async_copy(eattr_hbm.at[pl.ds(cb, CHUNK)], attr_v[b], gsem[b])

    def wait_gather_ei(k, b):
        cb = ebase + k * CHUNK
        pltpu.make_async_copy(ei_hbm.at[pl.ds(E + cb, CHUNK)], ei_v[b],
                              eisem[b]).wait()

    def wait_gather_attr(k, b):
        cb = ebase + k * CHUNK
        pltpu.make_async_copy(eattr_hbm.at[pl.ds(cb, CHUNK)], attr_v[b],
                              gsem[b]).wait()

    start_gather(0, 0)

    NB_IT = (N_NBLOCKS + NW - 1) // NW

    for it in range(NB_IT):
        @pl.when(w + it * NW < N_NBLOCKS)
        def _(it=it):
            base = (w + it * NW) * SUB
            pltpu.async_copy(batch_hbm.at[pl.ds(base, SUB)], nseg_v[it],
                             gsem1)
            pltpu.async_copy(x_hbm.at[pl.ds(base, SUB)], nstage[it], gsem1)

    @pl.when(w == NW - 1)
    def _():
        pltpu.async_copy(batch_hbm.at[pl.ds(NREM_BASE, NREM)], nrem_v, gsem1)
        pltpu.async_copy(x_hbm.at[pl.ds(NREM_BASE, NREM)],
                         xstage_v.at[pl.ds(0, NREM)], gsem1)

    zero16 = jnp.zeros((16,), jnp.float32)
    for t in range(B // 16):
        ncnt_v[pl.ds(t * 16, 16)] = zero16
        ecnt_v[pl.ds(t * 16, 16)] = zero16
    zstage = xstage_v.at[pl.ds(NREM, RPT)]
    for i in range(RPT):
        for j in range(D // 16):
            xstage_v[NREM + i, pl.ds(j * 16, 16)] = zero16

    pltpu.sync_copy(batch_hbm, batch_v)

    pltpu.sync_copy(zstage, nacc_s.at[pl.ds(s * RPT, RPT)])
    pltpu.sync_copy(zstage, eacc_s.at[pl.ds(s * RPT, RPT)])

    plsc.subcore_barrier()

    for it in range(NB_IT):
        @pl.when(w + it * NW < N_NBLOCKS)
        def _(it=it):
            base = (w + it * NW) * SUB
            pltpu.make_async_copy(batch_hbm.at[pl.ds(base, SUB)], nseg_v[it],
                                  gsem1).wait()
            pltpu.make_async_copy(x_hbm.at[pl.ds(base, SUB)], nstage[it],
                                  gsem1).wait()
            pltpu.async_copy(nstage[it], nacc_s.at[nseg_v[it]], ssem0,
                             add=True)
            for t in range(SUB // 16):
                _histo_update(ncnt_v, nseg_v[it][pl.ds(t * 16, 16)])

    @pl.when(w == NW - 1)
    def _():
        pltpu.make_async_copy(batch_hbm.at[pl.ds(NREM_BASE, NREM)], nrem_v,
                              gsem1).wait()
        pltpu.make_async_copy(x_hbm.at[pl.ds(NREM_BASE, NREM)],
                              xstage_v.at[pl.ds(0, NREM)], gsem1).wait()
        pltpu.async_copy(xstage_v.at[pl.ds(0, NREM)], nacc_s.at[nrem_v],
                         ssem0, add=True)
        _histo_update(ncnt_v, nrem_v[pl.ds(0, 16)])
        pltpu.make_async_copy(xstage_v.at[pl.ds(0, NREM)], nacc_s.at[nrem_v],
                              ssem0).wait()

    for it in range(NB_IT):
        @pl.when(w + it * NW < N_NBLOCKS)
        def _(it=it):
            pltpu.make_async_copy(nstage[it], nacc_s.at[nseg_v[it]],
                                  ssem0).wait()

    start_gather(1, 1)
    start_gather(2, 2)

    def compute_segs(b):
        for j in range(NSUB):
            for t in range(SUB // 16):
                idx = ei_v[b][pl.ds(j * SUB + t * 16, 16)]
                seg = plsc.load_gather(batch_v, [idx])
                seg_v[b][j][pl.ds(t * 16, 16)] = seg
                _histo_update(ecnt_v, seg)

    def issue_scatter(b):
        for j in range(NSUB):
            pltpu.async_copy(attr_v[b].at[pl.ds(j * SUB, SUB)],
                             eacc_s.at[seg_v[b][j]], ssem[b], add=True)

    def drain_scatter(b):
        for j in range(NSUB):
            pltpu.make_async_copy(attr_v[b].at[pl.ds(j * SUB, SUB)],
                                  eacc_s.at[seg_v[b][j]], ssem[b]).wait()

    def estep(k, b, drain_b, prefetch):
        wait_gather_ei(k, b)
        compute_segs(b)
        wait_gather_attr(k, b)
        issue_scatter(b)
        if drain_b is not None:
            drain_scatter(drain_b)
        if prefetch:
            start_gather(k + 2, drain_b)

    estep(0, 0, None, False)

    NTRI = (N_ECHUNKS - 3) // NBUF

    def etri(i, _):
        k = 3 * i + 1
        estep(k, 1, 0, True)
        estep(k + 1, 2, 1, True)
        estep(k + 2, 0, 2, True)
        return 0

    lax.fori_loop(0, NTRI, etri, 0)

    estep(N_ECHUNKS - 2, 1, 0, False)
    estep(N_ECHUNKS - 1, 2, 1, False)
    drain_scatter(2)

    @pl.when(w < EREM_BLOCKS)
    def _():
        base = EREM_BASE + w * SUB
        pltpu.sync_copy(ei_hbm.at[pl.ds(E + base, SUB)],
                        ei_v0.at[pl.ds(0, SUB)])
        pltpu.sync_copy(eattr_hbm.at[pl.ds(base, SUB)],
                        attr_v0.at[pl.ds(0, SUB)])
        for t in range(SUB // 16):
            idx = ei_v0[pl.ds(t * 16, 16)]
            seg = plsc.load_gather(batch_v, [idx])
            sg00[pl.ds(t * 16, 16)] = seg
            _histo_update(ecnt_v, seg)
        pltpu.sync_copy(attr_v0.at[pl.ds(0, SUB)], eacc_s.at[sg00], add=True)

    pltpu.sync_copy(ncnt_v, ncnt_out.at[c, s])
    pltpu.sync_copy(ecnt_v, ecnt_out.at[c, s])

    plsc.subcore_barrier()

    pltpu.sync_copy(nacc_s.at[pl.ds(s * RPT, RPT)],
                    nacc_out.at[c, pl.ds(s * RPT, RPT)])
    pltpu.sync_copy(eacc_s.at[pl.ds(s * RPT, RPT)],
                    eacc_out.at[c, pl.ds(s * RPT, RPT)])


_sc_segment_sums = functools.partial(
    pl.kernel,
    out_type=(
        jax.ShapeDtypeStruct((NC, B, D), jnp.float32),
        jax.ShapeDtypeStruct((NC, NS, B), jnp.float32),
        jax.ShapeDtypeStruct((NC, B, D), jnp.float32),
        jax.ShapeDtypeStruct((NC, NS, B), jnp.float32),
    ),
    mesh=plsc.VectorSubcoreMesh(core_axis_name="c", subcore_axis_name="s"),
    compiler_params=pltpu.CompilerParams(needs_layout_passes=False),
    scratch_types=[
        pltpu.VMEM((N,), jnp.int32),
        pltpu.VMEM((CHUNK,), jnp.int32),
        pltpu.VMEM((CHUNK,), jnp.int32),
        pltpu.VMEM((CHUNK,), jnp.int32),
        pltpu.VMEM((CHUNK, D), jnp.float32),
        pltpu.VMEM((CHUNK, D), jnp.float32),
        pltpu.VMEM((CHUNK, D), jnp.float32),
        pltpu.VMEM((NREM + B // NS, D), jnp.float32),
    ] + [pltpu.VMEM((SUB,), jnp.int32)] * 6
    + [
        pltpu.VMEM((NREM,), jnp.int32),
        pltpu.VMEM((B,), jnp.float32),
        pltpu.VMEM((B,), jnp.float32),
        pltpu.VMEM_SHARED((B, D), jnp.float32),
        pltpu.VMEM_SHARED((B, D), jnp.float32),
        pltpu.SemaphoreType.DMA,
        pltpu.SemaphoreType.DMA,
        pltpu.SemaphoreType.DMA,
        pltpu.SemaphoreType.DMA,
        pltpu.SemaphoreType.DMA,
        pltpu.SemaphoreType.DMA,
        pltpu.SemaphoreType.DMA,
        pltpu.SemaphoreType.DMA,
        pltpu.SemaphoreType.DMA,
    ],
)(_sc_body)


def _mlp_body(nacc, ncnt, eacc, ecnt, u, w1, b1, w2, b2, out):
    nsum = nacc[0] + nacc[1]
    esum = eacc[0] + eacc[1]
    ncount = jnp.sum(ncnt[...], axis=(0, 1)).reshape(B, 1)
    ecount = jnp.sum(ecnt[...], axis=(0, 1)).reshape(B, 1)
    nmean = nsum / jnp.maximum(ncount, 1.0)
    emean = esum / jnp.maximum(ecount, 1.0)
    U = w1.shape[0] - 2 * D
    h = (jnp.dot(u[...], w1[0:U], preferred_element_type=jnp.float32)
         + jnp.dot(nmean, w1[U:U + D], preferred_element_type=jnp.float32)
         + jnp.dot(emean, w1[U + D:], preferred_element_type=jnp.float32)
         + b1[...])
    h = jnp.maximum(h, 0.0)
    out[...] = jnp.dot(h, w2[...], preferred_element_type=jnp.float32) + b2[...]


def kernel(x, edge_index, edge_attr, u, batch, W1, b1, W2, b2):
    ei = edge_index if edge_index.dtype == jnp.int32 else edge_index.astype(jnp.int32)
    ei = ei.reshape(-1)
    batch_i = batch if batch.dtype == jnp.int32 else batch.astype(jnp.int32)
    nacc, ncnt, eacc, ecnt = _sc_segment_sums(ei, x, batch_i, edge_attr)

    out = pl.pallas_call(
        _mlp_body,
        out_shape=jax.ShapeDtypeStruct((B, D), jnp.float32),
    )(nacc, ncnt, eacc, ecnt, u, W1,
      b1.reshape(1, D), W2, b2.reshape(1, D))
    return out
